# baseline (speedup 1.0000x reference)
import functools

import jax
import jax.numpy as jnp
from jax import lax
from jax.experimental import pallas as pl
from jax.experimental.pallas import tpu as pltpu
from jax.experimental.pallas import tpu_sc as plsc

B = 16384
VOCAB = 1000000
EMBED_DIM = 32
NUM_CONT = 13

_info = plsc.get_sparse_core_info()
NC, NS, L = _info.num_cores, _info.num_subcores, _info.num_lanes
NW = NC * NS
BPW = B // NW
NGRP = BPW // L

BLK = 65536
_GRID = (VOCAB + BLK - 1) // BLK
BLKB = 2048
_ZSTEPS = B // BLKB


_RSPLIT = 4
_RB = EMBED_DIM // _RSPLIT


def _dense_body(t_ref, w_ref, x_ref, wcb_ref, y_ref, z_ref):
    i = pl.program_id(0)
    j = pl.program_id(1)
    part = jax.lax.dot_general(
        w_ref[...], t_ref[...], (((0,), (0,)), ((), ())),
        preferred_element_type=jnp.float32)[0]

    @pl.when(j == 0)
    def _():
        y_ref[...] = part

    @pl.when(j > 0)
    def _():
        y_ref[...] = y_ref[...] + part

    @pl.when(jnp.logical_and(i < _ZSTEPS, j == 0))
    def _():
        z_ref[...] = jax.lax.dot_general(
            x_ref[...], wcb_ref[:NUM_CONT, :], (((1,), (0,)), ((), ())),
            preferred_element_type=jnp.float32)[:, 0] + wcb_ref[NUM_CONT, 0]


_dense = pl.pallas_call(
    _dense_body,
    grid=(_GRID, _RSPLIT),
    in_specs=[
        pl.BlockSpec((_RB, BLK), lambda i, j: (j, i)),
        pl.BlockSpec((_RB, 1), lambda i, j: (j, 0)),
        pl.BlockSpec((BLKB, NUM_CONT),
                     lambda i, j: (jnp.minimum(i, _ZSTEPS - 1), 0)),
        pl.BlockSpec((NUM_CONT + 1, 1), lambda i, j: (0, 0)),
    ],
    out_specs=[
        pl.BlockSpec((BLK,), lambda i, j: (i,)),
        pl.BlockSpec((BLKB,), lambda i, j: (jnp.minimum(i, _ZSTEPS - 1),)),
    ],
    out_shape=[
        jax.ShapeDtypeStruct((VOCAB,), jnp.float32),
        jax.ShapeDtypeStruct((B,), jnp.float32),
    ],
)

_mesh = plsc.VectorSubcoreMesh(core_axis_name="c", subcore_axis_name="s")


@functools.partial(
    pl.kernel,
    mesh=_mesh,
    out_type=jax.ShapeDtypeStruct((B,), jnp.float32),
    scratch_types=[
        pltpu.VMEM((BPW,), jnp.int32),
        pltpu.VMEM((BPW,), jnp.float32),
        pltpu.VMEM((BPW,), jnp.float32),
        pltpu.VMEM((BPW,), jnp.float32),
        pltpu.SemaphoreType.DMA,
    ],
    compiler_params=pltpu.CompilerParams(needs_layout_passes=False),
)
def _sc_lookup(idx_hbm, y_hbm, z_hbm, out_hbm, idx_v, y_v, z_v, out_v, sem):
    wid = lax.axis_index("s") * NC + lax.axis_index("c")
    base = wid * BPW
    pltpu.sync_copy(idx_hbm.at[pl.ds(base, BPW)], idx_v)
    gather = pltpu.async_copy(y_hbm.at[idx_v], y_v, sem)
    pltpu.sync_copy(z_hbm.at[pl.ds(base, BPW)], z_v)
    gather.wait()

    def body(g, carry):
        row0 = g * L
        out_v[pl.ds(row0, L)] = y_v[pl.ds(row0, L)] + z_v[pl.ds(row0, L)]
        return carry

    lax.fori_loop(0, NGRP, body, 0)
    pltpu.sync_copy(out_v, out_hbm.at[pl.ds(base, BPW)])


def kernel(x_cat, x_cont, emb_table, fc_w, fc_b):
    table_t = emb_table.T
    w_col = fc_w[:EMBED_DIM]
    wcb = jnp.concatenate([fc_w[EMBED_DIM:, 0], fc_b]).reshape(NUM_CONT + 1, 1)
    y, z = _dense(table_t, w_col, x_cont, wcb)
    idx = x_cat.reshape(B)
    out = _sc_lookup(idx, y, z)
    return out.reshape(B, 1)


# speedup vs baseline: 1.5143x; 1.5143x over previous
import functools

import jax
import jax.numpy as jnp
from jax import lax
from jax.experimental import pallas as pl
from jax.experimental.pallas import tpu as pltpu
from jax.experimental.pallas import tpu_sc as plsc

B = 16384
VOCAB = 1000000
EMBED_DIM = 32
NUM_CONT = 13

_info = plsc.get_sparse_core_info()
NC, NS, L = _info.num_cores, _info.num_subcores, _info.num_lanes
NW = NC * NS
BPW = B // NW
NGRP = BPW // L

BLK = 65536
_GRID = (VOCAB + BLK - 1) // BLK
BLKB = 2048
_ZSTEPS = B // BLKB


_NSTREAM = 4
_RS = EMBED_DIM // _NSTREAM


def _dense_body(t0_ref, t1_ref, t2_ref, t3_ref, w_ref, x_ref, wcb_ref,
                y_ref, z_ref):
    i = pl.program_id(0)
    acc = jax.lax.dot_general(
        w_ref[pl.ds(0, _RS), :], t0_ref[...], (((0,), (0,)), ((), ())),
        preferred_element_type=jnp.float32)[0]
    for k, t_ref in enumerate((t1_ref, t2_ref, t3_ref)):
        acc = acc + jax.lax.dot_general(
            w_ref[pl.ds((k + 1) * _RS, _RS), :], t_ref[...],
            (((0,), (0,)), ((), ())),
            preferred_element_type=jnp.float32)[0]
    y_ref[...] = acc

    @pl.when(i < _ZSTEPS)
    def _():
        z_ref[...] = jax.lax.dot_general(
            x_ref[...], wcb_ref[:NUM_CONT, :], (((1,), (0,)), ((), ())),
            preferred_element_type=jnp.float32)[:, 0] + wcb_ref[NUM_CONT, 0]


_dense = pl.pallas_call(
    _dense_body,
    grid=(_GRID,),
    in_specs=[
        pl.BlockSpec((_RS, BLK), lambda i: (0, i)),
        pl.BlockSpec((_RS, BLK), lambda i: (1, i)),
        pl.BlockSpec((_RS, BLK), lambda i: (2, i)),
        pl.BlockSpec((_RS, BLK), lambda i: (3, i)),
        pl.BlockSpec((EMBED_DIM, 1), lambda i: (0, 0)),
        pl.BlockSpec((BLKB, NUM_CONT), lambda i: (jnp.minimum(i, _ZSTEPS - 1), 0)),
        pl.BlockSpec((NUM_CONT + 1, 1), lambda i: (0, 0)),
    ],
    out_specs=[
        pl.BlockSpec((BLK,), lambda i: (i,)),
        pl.BlockSpec((BLKB,), lambda i: (jnp.minimum(i, _ZSTEPS - 1),)),
    ],
    out_shape=[
        jax.ShapeDtypeStruct((VOCAB,), jnp.float32),
        jax.ShapeDtypeStruct((B,), jnp.float32),
    ],
)

_mesh = plsc.VectorSubcoreMesh(core_axis_name="c", subcore_axis_name="s")


@functools.partial(
    pl.kernel,
    mesh=_mesh,
    out_type=jax.ShapeDtypeStruct((B,), jnp.float32),
    scratch_types=[
        pltpu.VMEM((BPW,), jnp.int32),
        pltpu.VMEM((BPW,), jnp.float32),
        pltpu.VMEM((BPW,), jnp.float32),
        pltpu.VMEM((BPW,), jnp.float32),
        pltpu.SemaphoreType.DMA,
    ],
    compiler_params=pltpu.CompilerParams(needs_layout_passes=False),
)
def _sc_lookup(idx_hbm, y_hbm, z_hbm, out_hbm, idx_v, y_v, z_v, out_v, sem):
    wid = lax.axis_index("s") * NC + lax.axis_index("c")
    base = wid * BPW
    pltpu.sync_copy(idx_hbm.at[pl.ds(base, BPW)], idx_v)
    gather = pltpu.async_copy(y_hbm.at[idx_v], y_v, sem)
    pltpu.sync_copy(z_hbm.at[pl.ds(base, BPW)], z_v)
    gather.wait()

    def body(g, carry):
        row0 = g * L
        out_v[pl.ds(row0, L)] = y_v[pl.ds(row0, L)] + z_v[pl.ds(row0, L)]
        return carry

    lax.fori_loop(0, NGRP, body, 0)
    pltpu.sync_copy(out_v, out_hbm.at[pl.ds(base, BPW)])


def kernel(x_cat, x_cont, emb_table, fc_w, fc_b):
    table_t = emb_table.T
    w_col = fc_w[:EMBED_DIM]
    wcb = jnp.concatenate([fc_w[EMBED_DIM:, 0], fc_b]).reshape(NUM_CONT + 1, 1)
    y, z = _dense(table_t, table_t, table_t, table_t, w_col, x_cont, wcb)
    idx = x_cat.reshape(B)
    out = _sc_lookup(idx, y, z)
    return out.reshape(B, 1)


# speedup vs baseline: 1.5858x; 1.0472x over previous
import functools

import jax
import jax.numpy as jnp
from jax import lax
from jax.experimental import pallas as pl
from jax.experimental.pallas import tpu as pltpu
from jax.experimental.pallas import tpu_sc as plsc

B = 16384
VOCAB = 1000000
EMBED_DIM = 32
NUM_CONT = 13

_info = plsc.get_sparse_core_info()
NC, NS, L = _info.num_cores, _info.num_subcores, _info.num_lanes
NW = NC * NS
BPW = B // NW
NGRP = BPW // L

BLK = 65536
_GRID = (VOCAB + BLK - 1) // BLK
BLKB = 2048
_ZSTEPS = B // BLKB


def _dense_body(t_ref, w_ref, x_ref, wcb_ref, y_ref, z_ref):
    i = pl.program_id(0)
    y_ref[...] = jax.lax.dot_general(
        w_ref[...], t_ref[...], (((0,), (0,)), ((), ())),
        preferred_element_type=jnp.float32)[0]

    @pl.when(i < _ZSTEPS)
    def _():
        z_ref[...] = jax.lax.dot_general(
            x_ref[...], wcb_ref[:NUM_CONT, :], (((1,), (0,)), ((), ())),
            preferred_element_type=jnp.float32)[:, 0] + wcb_ref[NUM_CONT, 0]


_dense = pl.pallas_call(
    _dense_body,
    grid=(_GRID,),
    in_specs=[
        pl.BlockSpec((EMBED_DIM, BLK), lambda i: (0, i)),
        pl.BlockSpec((EMBED_DIM, 1), lambda i: (0, 0)),
        pl.BlockSpec((BLKB, NUM_CONT), lambda i: (jnp.minimum(i, _ZSTEPS - 1), 0)),
        pl.BlockSpec((NUM_CONT + 1, 1), lambda i: (0, 0)),
    ],
    out_specs=[
        pl.BlockSpec((BLK,), lambda i: (i,)),
        pl.BlockSpec((BLKB,), lambda i: (jnp.minimum(i, _ZSTEPS - 1),)),
    ],
    out_shape=[
        jax.ShapeDtypeStruct((VOCAB,), jnp.float32),
        jax.ShapeDtypeStruct((B,), jnp.float32),
    ],
)

_mesh = plsc.VectorSubcoreMesh(core_axis_name="c", subcore_axis_name="s")


@functools.partial(
    pl.kernel,
    mesh=_mesh,
    out_type=jax.ShapeDtypeStruct((B,), jnp.float32),
    scratch_types=[
        pltpu.VMEM((BPW,), jnp.int32),
        pltpu.VMEM((BPW,), jnp.float32),
        pltpu.VMEM((BPW,), jnp.float32),
        pltpu.VMEM((BPW,), jnp.float32),
        pltpu.SemaphoreType.DMA,
    ],
    compiler_params=pltpu.CompilerParams(
        needs_layout_passes=False, skip_device_barrier=True),
)
def _sc_lookup(idx_hbm, y_hbm, z_hbm, out_hbm, idx_v, y_v, z_v, out_v, sem):
    wid = lax.axis_index("s") * NC + lax.axis_index("c")
    base = wid * BPW
    pltpu.sync_copy(idx_hbm.at[pl.ds(base, BPW)], idx_v)
    gather = pltpu.async_copy(y_hbm.at[idx_v], y_v, sem)
    pltpu.sync_copy(z_hbm.at[pl.ds(base, BPW)], z_v)
    gather.wait()

    def body(g, carry):
        row0 = g * L
        out_v[pl.ds(row0, L)] = y_v[pl.ds(row0, L)] + z_v[pl.ds(row0, L)]
        return carry

    lax.fori_loop(0, NGRP, body, 0)
    pltpu.sync_copy(out_v, out_hbm.at[pl.ds(base, BPW)])


def kernel(x_cat, x_cont, emb_table, fc_w, fc_b):
    table_t = emb_table.T
    w_col = fc_w[:EMBED_DIM]
    wcb = jnp.concatenate([fc_w[EMBED_DIM:, 0], fc_b]).reshape(NUM_CONT + 1, 1)
    y, z = _dense(table_t, w_col, x_cont, wcb)
    idx = x_cat.reshape(B)
    out = _sc_lookup(idx, y, z)
    return out.reshape(B, 1)
